# Initial kernel scaffold; baseline (speedup 1.0000x reference)
import jax
import jax.numpy as jnp
from jax.experimental import pallas as pl


def kernel(emb, edge_index):
    raise NotImplementedError("write your pallas kernel here")


# speedup vs baseline: 19.4871x; 19.4871x over previous
import dataclasses

import jax
import jax.numpy as jnp
from jax import lax
from jax.experimental import pallas as pl
from jax.experimental.pallas import tpu as pltpu
from jax.experimental.pallas import tpu_sc as plsc

N = 100000
H = 16
NP = 100352
E = 1600000
EPT = 100352
EPAD = EPT * 16
NWIN = 196
WROWS = EPT // 128
CHUNK = NP // 16
EW = 224
NLAYERS = 3


def _rsqrt16(d):
    i = plsc.bitcast(d, jnp.int32)
    i = jnp.int32(0x5F3759DF) - lax.shift_right_logical(i, 1)
    y = plsc.bitcast(i, jnp.float32)
    for _ in range(3):
        y = y * (1.5 - 0.5 * d * y * y)
    return jnp.where(d > 0.0, y, 0.0)


def _bcast16(ref, i):
    return plsc.load_gather(ref, [jnp.full((16,), i, jnp.int32)])


def _sc_body(emb_f, row2, col2, out,
             y_f, sum_f,
             accum, dd,
             zb, ab, sbuf, dw, ones, idxr, idxc, vals,
             gsem):
    c = lax.axis_index("c")
    t = lax.axis_index("s")
    r0 = t * CHUNK
    w0 = t * WROWS
    hoff = c * NP

    z16 = jnp.zeros((16,), jnp.float32)

    @pl.loop(0, EW)
    def _(i):
        zb[i, :] = z16

    @pl.loop(0, EW, step=16)
    def _(i):
        dw[pl.ds(i, 16)] = z16

    @pl.loop(0, 128, step=16)
    def _(i):
        ones[pl.ds(i, 16)] = jnp.ones((16,), jnp.float32)

    @pl.loop(0, CHUNK, step=EW)
    def _(w):
        pltpu.sync_copy(zb, accum.at[pl.ds(r0 + w, EW), :])
        pltpu.sync_copy(dw, dd.at[pl.ds(r0 + w, EW)])

    plsc.subcore_barrier()

    @pl.loop(0, NWIN)
    def _(w):
        pltpu.sync_copy(row2.at[pl.ds(w0 + w * 4, 4), :], idxr)
        for j in range(4):
            pltpu.sync_copy(ones, dd.at[idxr.at[j]], add=True)

    plsc.subcore_barrier()

    @pl.loop(0, CHUNK, step=EW)
    def _(w):
        g0 = r0 + w
        pltpu.sync_copy(dd.at[pl.ds(g0, EW)], dw)

        @pl.loop(0, EW, step=16)
        def _(i):
            dw[pl.ds(i, 16)] = _rsqrt16(dw[pl.ds(i, 16)])

        pltpu.sync_copy(dw, dd.at[pl.ds(g0, EW)])
        pltpu.sync_copy(emb_f.at[pl.ds(hoff + g0, EW), :], ab)

        @pl.loop(0, EW)
        def _(i):
            ab[i, :] = ab[i, :] * _bcast16(dw, i)

        pltpu.sync_copy(ab, y_f.at[pl.ds(hoff + g0, EW), :])

    plsc.subcore_barrier()

    for l in range(NLAYERS):
        @pl.loop(0, NWIN)
        def _(w):
            base = w0 + w * 4
            pltpu.sync_copy(col2.at[pl.ds(base, 4), :], idxc)
            pltpu.sync_copy(row2.at[pl.ds(base, 4), :], idxr)
            for j in range(4):
                @pl.loop(0, 128, step=16)
                def _(i):
                    idxc[j, pl.ds(i, 16)] = idxc[j, pl.ds(i, 16)] + hoff
            cps = [
                pltpu.async_copy(y_f.at[idxc.at[j]],
                                 vals.at[pl.ds(j * 128, 128), :], gsem)
                for j in range(4)
            ]
            for j in range(4):
                cps[j].wait()
                pltpu.sync_copy(vals.at[pl.ds(j * 128, 128), :],
                                accum.at[idxr.at[j]], add=True)

        plsc.subcore_barrier()

        @pl.loop(0, CHUNK, step=EW)
        def _(w):
            g0 = r0 + w
            pltpu.sync_copy(accum.at[pl.ds(g0, EW), :], ab)
            pltpu.sync_copy(zb, accum.at[pl.ds(g0, EW), :])
            pltpu.sync_copy(dd.at[pl.ds(g0, EW)], dw)
            if l == 0:
                pltpu.sync_copy(emb_f.at[pl.ds(hoff + g0, EW), :], sbuf)
            else:
                pltpu.sync_copy(sum_f.at[pl.ds(hoff + g0, EW), :], sbuf)

            if l < NLAYERS - 1:
                @pl.loop(0, EW)
                def _(i):
                    d = _bcast16(dw, i)
                    x = ab[i, :] * d
                    sbuf[i, :] = sbuf[i, :] + x
                    ab[i, :] = x * d
                pltpu.sync_copy(sbuf, sum_f.at[pl.ds(hoff + g0, EW), :])
                pltpu.sync_copy(ab, y_f.at[pl.ds(hoff + g0, EW), :])
            else:
                @pl.loop(0, EW)
                def _(i):
                    x = ab[i, :] * _bcast16(dw, i)
                    sbuf[i, :] = (sbuf[i, :] + x) * 0.25
                pltpu.sync_copy(sbuf, out.at[pl.ds(hoff + g0, EW), :])

        plsc.subcore_barrier()


@jax.jit
def _lightgcn_sc(emb_f, row2, col2):
    cp = pltpu.CompilerParams(use_tc_tiling_on_sc=False)
    if "needs_layout_passes" in pltpu.CompilerParams.__dataclass_fields__:
        cp = dataclasses.replace(cp, needs_layout_passes=False)
    mesh = plsc.VectorSubcoreMesh(core_axis_name="c", subcore_axis_name="s")
    k = pl.kernel(
        _sc_body,
        out_type=jax.ShapeDtypeStruct((2 * NP, H), jnp.float32),
        mesh=mesh,
        scratch_types=[
            pltpu.HBM((2 * NP, H), jnp.float32),
            pltpu.HBM((2 * NP, H), jnp.float32),
            pltpu.VMEM_SHARED((NP, H), jnp.float32),
            pltpu.VMEM_SHARED((NP,), jnp.float32),
            pltpu.VMEM((EW, H), jnp.float32),
            pltpu.VMEM((EW, H), jnp.float32),
            pltpu.VMEM((EW, H), jnp.float32),
            pltpu.VMEM((EW,), jnp.float32),
            pltpu.VMEM((128,), jnp.float32),
            pltpu.VMEM((4, 128), jnp.int32),
            pltpu.VMEM((4, 128), jnp.int32),
            pltpu.VMEM((512, H), jnp.float32),
            pltpu.SemaphoreType.DMA,
        ],
        compiler_params=cp,
    )
    return k(emb_f, row2, col2)


def kernel(emb, edge_index):
    emb = emb.astype(jnp.float32)
    row = edge_index[0].astype(jnp.int32)
    col = edge_index[1].astype(jnp.int32)
    npad = EPAD - E
    ar = jnp.arange(npad, dtype=jnp.int32)
    row_p = jnp.concatenate([row, N + ar % (NP - N)]).reshape(-1, 128)
    col_p = jnp.concatenate([col, ar % 2048]).reshape(-1, 128)
    emb_f = jnp.zeros((2 * NP, H), jnp.float32)
    emb_f = emb_f.at[:N].set(emb[:, :H]).at[NP:NP + N].set(emb[:, H:])
    outp = _lightgcn_sc(emb_f, row_p, col_p)
    final = jnp.concatenate([outp[:N], outp[NP:NP + N]], axis=1)
    return final[:40000], final[40000:90000], final[90000:]


# speedup vs baseline: 22.6386x; 1.1617x over previous
import dataclasses

import jax
import jax.numpy as jnp
from jax import lax
from jax.experimental import pallas as pl
from jax.experimental.pallas import tpu as pltpu
from jax.experimental.pallas import tpu_sc as plsc

N = 100000
H = 16
NP = 100352
E = 1600000
WSZ = 384
NWIN = 262
EPT = WSZ * NWIN
EPAD = EPT * 16
WROWS = EPT // 128
CHUNK = NP // 16
EW = 112
NLAYERS = 3


def _rsqrt16(d):
    i = plsc.bitcast(d, jnp.int32)
    i = jnp.int32(0x5F3759DF) - lax.shift_right_logical(i, 1)
    y = plsc.bitcast(i, jnp.float32)
    for _ in range(3):
        y = y * (1.5 - 0.5 * d * y * y)
    return jnp.where(d > 0.0, y, 0.0)


def _bcast16(ref, i):
    return plsc.load_gather(ref, [jnp.full((16,), i, jnp.int32)])


def _sc_body(emb_f, row2, col3, out,
             y_f, sum_f,
             accum, dd,
             zb, ab, sbuf, dw, ones,
             ir0, ir1, ic0, ic1, v0, v1,
             semA, semB):
    c = lax.axis_index("c")
    t = lax.axis_index("s")
    r0 = t * CHUNK
    w0 = t * WROWS
    hoff = c * NP

    z16 = jnp.zeros((16,), jnp.float32)

    @pl.loop(0, EW)
    def _(i):
        zb[i, :] = z16

    @pl.loop(0, EW, step=16)
    def _(i):
        dw[pl.ds(i, 16)] = z16

    @pl.loop(0, 128, step=16)
    def _(i):
        ones[pl.ds(i, 16)] = jnp.ones((16,), jnp.float32)

    @pl.loop(0, CHUNK, step=EW)
    def _(w):
        pltpu.sync_copy(zb, accum.at[pl.ds(r0 + w, EW), :])
        pltpu.sync_copy(dw, dd.at[pl.ds(r0 + w, EW)])

    plsc.subcore_barrier()

    def _deg_drain(ir, sem):
        pltpu.make_async_copy(row2.at[pl.ds(0, 3), :], ir, sem).wait()
        for j in range(3):
            pltpu.sync_copy(ones, dd.at[ir.at[j]], add=True)

    pltpu.async_copy(row2.at[pl.ds(w0, 3), :], ir0, semA)

    @pl.loop(0, NWIN, step=2)
    def _(w):
        pltpu.async_copy(row2.at[pl.ds(w0 + (w + 1) * 3, 3), :], ir1, semB)
        _deg_drain(ir0, semA)

        @pl.when(w + 2 < NWIN)
        def _():
            pltpu.async_copy(row2.at[pl.ds(w0 + (w + 2) * 3, 3), :], ir0,
                             semA)

        _deg_drain(ir1, semB)

    plsc.subcore_barrier()

    @pl.loop(0, CHUNK, step=EW)
    def _(w):
        g0 = r0 + w
        pltpu.sync_copy(dd.at[pl.ds(g0, EW)], dw)

        @pl.loop(0, EW, step=16)
        def _(i):
            dw[pl.ds(i, 16)] = _rsqrt16(dw[pl.ds(i, 16)])

        pltpu.sync_copy(dw, dd.at[pl.ds(g0, EW)])
        pltpu.sync_copy(emb_f.at[pl.ds(hoff + g0, EW), :], ab)

        @pl.loop(0, EW)
        def _(i):
            ab[i, :] = ab[i, :] * _bcast16(dw, i)

        pltpu.sync_copy(ab, y_f.at[pl.ds(hoff + g0, EW), :])

    plsc.subcore_barrier()

    def _load_fire(w, ir, ic, v, sem):
        base = w0 + w * 3
        pltpu.sync_copy(row2.at[pl.ds(base, 3), :], ir)
        pltpu.sync_copy(col3.at[c, pl.ds(base, 3), :], ic)
        for j in range(3):
            pltpu.async_copy(y_f.at[ic.at[j]],
                             v.at[pl.ds(j * 128, 128), :], sem)

    def _drain(ir, v, sem):
        for j in range(3):
            pltpu.make_async_copy(y_f.at[pl.ds(0, 128), :],
                                  v.at[pl.ds(j * 128, 128), :], sem).wait()
        for j in range(3):
            pltpu.sync_copy(v.at[pl.ds(j * 128, 128), :],
                            accum.at[ir.at[j]], add=True)

    for l in range(NLAYERS):
        _load_fire(0, ir0, ic0, v0, semA)

        @pl.loop(0, NWIN, step=2)
        def _(w):
            _load_fire(w + 1, ir1, ic1, v1, semB)
            _drain(ir0, v0, semA)

            @pl.when(w + 2 < NWIN)
            def _():
                _load_fire(w + 2, ir0, ic0, v0, semA)

            _drain(ir1, v1, semB)

        plsc.subcore_barrier()

        @pl.loop(0, CHUNK, step=EW)
        def _(w):
            g0 = r0 + w
            pltpu.sync_copy(accum.at[pl.ds(g0, EW), :], ab)
            pltpu.sync_copy(zb, accum.at[pl.ds(g0, EW), :])
            pltpu.sync_copy(dd.at[pl.ds(g0, EW)], dw)
            if l == 0:
                pltpu.sync_copy(emb_f.at[pl.ds(hoff + g0, EW), :], sbuf)
            else:
                pltpu.sync_copy(sum_f.at[pl.ds(hoff + g0, EW), :], sbuf)

            if l < NLAYERS - 1:
                @pl.loop(0, EW)
                def _(i):
                    d = _bcast16(dw, i)
                    x = ab[i, :] * d
                    sbuf[i, :] = sbuf[i, :] + x
                    ab[i, :] = x * d
                pltpu.sync_copy(sbuf, sum_f.at[pl.ds(hoff + g0, EW), :])
                pltpu.sync_copy(ab, y_f.at[pl.ds(hoff + g0, EW), :])
            else:
                @pl.loop(0, EW)
                def _(i):
                    x = ab[i, :] * _bcast16(dw, i)
                    sbuf[i, :] = (sbuf[i, :] + x) * 0.25
                pltpu.sync_copy(sbuf, out.at[pl.ds(hoff + g0, EW), :])

        plsc.subcore_barrier()


@jax.jit
def _lightgcn_sc(emb_f, row2, col3):
    cp = pltpu.CompilerParams(use_tc_tiling_on_sc=False)
    if "needs_layout_passes" in pltpu.CompilerParams.__dataclass_fields__:
        cp = dataclasses.replace(cp, needs_layout_passes=False)
    mesh = plsc.VectorSubcoreMesh(core_axis_name="c", subcore_axis_name="s")
    k = pl.kernel(
        _sc_body,
        out_type=jax.ShapeDtypeStruct((2 * NP, H), jnp.float32),
        mesh=mesh,
        scratch_types=[
            pltpu.HBM((2 * NP, H), jnp.float32),
            pltpu.HBM((2 * NP, H), jnp.float32),
            pltpu.VMEM_SHARED((NP, H), jnp.float32),
            pltpu.VMEM_SHARED((NP,), jnp.float32),
            pltpu.VMEM((EW, H), jnp.float32),
            pltpu.VMEM((EW, H), jnp.float32),
            pltpu.VMEM((EW, H), jnp.float32),
            pltpu.VMEM((EW,), jnp.float32),
            pltpu.VMEM((128,), jnp.float32),
            pltpu.VMEM((3, 128), jnp.int32),
            pltpu.VMEM((3, 128), jnp.int32),
            pltpu.VMEM((3, 128), jnp.int32),
            pltpu.VMEM((3, 128), jnp.int32),
            pltpu.VMEM((WSZ, H), jnp.float32),
            pltpu.VMEM((WSZ, H), jnp.float32),
            pltpu.SemaphoreType.DMA,
            pltpu.SemaphoreType.DMA,
        ],
        compiler_params=cp,
    )
    return k(emb_f, row2, col3)


def kernel(emb, edge_index):
    emb = emb.astype(jnp.float32)
    row = edge_index[0].astype(jnp.int32)
    col = edge_index[1].astype(jnp.int32)
    npad = EPAD - E
    ar = jnp.arange(npad, dtype=jnp.int32)
    pad_idx = N + ar % (NP - N)
    row_p = jnp.concatenate([row, pad_idx]).reshape(-1, 128)
    col_p = jnp.concatenate([col, pad_idx]).reshape(-1, 128)
    col3 = jnp.stack([col_p, col_p + NP])
    emb_f = jnp.zeros((2 * NP, H), jnp.float32)
    emb_f = emb_f.at[:N].set(emb[:, :H]).at[NP:NP + N].set(emb[:, H:])
    outp = _lightgcn_sc(emb_f, row_p, col3)
    final = jnp.concatenate([outp[:N], outp[NP:NP + N]], axis=1)
    return final[:40000], final[40000:90000], final[90000:]


# speedup vs baseline: 28.5395x; 1.2607x over previous
import dataclasses

import jax
import jax.numpy as jnp
from jax import lax
from jax.experimental import pallas as pl
from jax.experimental.pallas import tpu as pltpu
from jax.experimental.pallas import tpu_sc as plsc

N = 100000
H = 16
NP = 100352
E = 1600000
WSZ = 384
NWIN = 264
EPT = WSZ * NWIN
EPAD = EPT * 16
CHUNK = NP // 16
EW = 112
NLAYERS = 3


def _rsqrt16(d):
    i = plsc.bitcast(d, jnp.int32)
    i = jnp.int32(0x5F3759DF) - lax.shift_right_logical(i, 1)
    y = plsc.bitcast(i, jnp.float32)
    for _ in range(3):
        y = y * (1.5 - 0.5 * d * y * y)
    return jnp.where(d > 0.0, y, 0.0)


def _bcast16(ref, i):
    return plsc.load_gather(ref, [jnp.full((16,), i, jnp.int32)])


def _sc_body(emb_f, rc3, out,
             y_f, sum_f, hdum,
             accum, dd,
             zb, ones,
             ab0, ab1, dw0, dw1, sb0, sb1,
             ic0, ic1, ic2, v0, v1,
             semA, semB, semI0, semI1, semI2,
             semEA, semEB, semWA, semWB):
    c = lax.axis_index("c")
    t = lax.axis_index("s")
    r0 = t * CHUNK
    i0 = t * (NWIN * 6)
    hoff = c * NP

    ics = (ic0, ic1, ic2)
    semIs = (semI0, semI1, semI2)
    vs = (v0, v1)
    semVs = (semA, semB)

    z16 = jnp.zeros((16,), jnp.float32)

    @pl.loop(0, EW)
    def _(i):
        zb[i, :] = z16

    @pl.loop(0, EW, step=16)
    def _(i):
        dw0[pl.ds(i, 16)] = z16

    @pl.loop(0, 128, step=16)
    def _(i):
        ones[pl.ds(i, 16)] = jnp.ones((16,), jnp.float32)

    @pl.loop(0, CHUNK, step=EW)
    def _(w):
        pltpu.sync_copy(zb, accum.at[pl.ds(r0 + w, EW), :])
        pltpu.sync_copy(dw0, dd.at[pl.ds(r0 + w, EW)])

    plsc.subcore_barrier()

    def _idx_fire(w, b):
        pltpu.async_copy(rc3.at[c, pl.ds(i0 + w * 6, 6), :], ics[b],
                         semIs[b])

    def _idx_wait(b):
        pltpu.make_async_copy(rc3.at[c, pl.ds(0, 6), :], ics[b],
                              semIs[b]).wait()

    _idx_fire(0, 0)

    @pl.loop(0, NWIN, step=2)
    def _(w):
        _idx_fire(w + 1, 1)
        _idx_wait(0)
        for j in range(3, 6):
            pltpu.sync_copy(ones, dd.at[ic0.at[j]], add=True)

        @pl.when(w + 2 < NWIN)
        def _():
            _idx_fire(w + 2, 0)

        _idx_wait(1)
        for j in range(3, 6):
            pltpu.sync_copy(ones, dd.at[ic1.at[j]], add=True)

    plsc.subcore_barrier()

    @pl.loop(0, CHUNK, step=EW)
    def _(w):
        g0 = r0 + w
        pltpu.sync_copy(dd.at[pl.ds(g0, EW)], dw0)

        @pl.loop(0, EW, step=16)
        def _(i):
            dw0[pl.ds(i, 16)] = _rsqrt16(dw0[pl.ds(i, 16)])

        pltpu.sync_copy(dw0, dd.at[pl.ds(g0, EW)])
        pltpu.sync_copy(emb_f.at[pl.ds(hoff + g0, EW), :], ab0)

        @pl.loop(0, EW)
        def _(i):
            ab0[i, :] = ab0[i, :] * _bcast16(dw0, i)

        pltpu.sync_copy(ab0, y_f.at[pl.ds(hoff + g0, EW), :])

    plsc.subcore_barrier()

    def _gather_fire(b, p):
        for j in range(3):
            pltpu.async_copy(y_f.at[ics[b].at[j]],
                             vs[p].at[pl.ds(j * 128, 128), :], semVs[p])

    def _drain(b, p):
        for j in range(3):
            pltpu.make_async_copy(y_f.at[pl.ds(0, 128), :],
                                  vs[p].at[pl.ds(j * 128, 128), :],
                                  semVs[p]).wait()
        for j in range(3):
            pltpu.sync_copy(vs[p].at[pl.ds(j * 128, 128), :],
                            accum.at[ics[b].at[j + 3]], add=True)

    def _epi_fire_in(l, w, ab, dw, sb, sem):
        g0 = r0 + w
        pltpu.async_copy(accum.at[pl.ds(g0, EW), :], ab, sem)
        pltpu.async_copy(dd.at[pl.ds(g0, EW)], dw, sem)
        src = emb_f if l == 0 else sum_f
        pltpu.async_copy(src.at[pl.ds(hoff + g0, EW), :], sb, sem)

    def _epi_wait_in(l, ab, dw, sb, sem):
        pltpu.make_async_copy(sum_f.at[pl.ds(0, EW), :], ab, sem).wait()
        pltpu.make_async_copy(hdum, dw, sem).wait()
        pltpu.make_async_copy(sum_f.at[pl.ds(0, EW), :], sb, sem).wait()

    def _epi_compute_write(l, w, ab, dw, sb, semw):
        g0 = r0 + w
        pltpu.sync_copy(zb, accum.at[pl.ds(g0, EW), :])
        if l < NLAYERS - 1:
            @pl.loop(0, EW)
            def _(i):
                d = _bcast16(dw, i)
                x = ab[i, :] * d
                sb[i, :] = sb[i, :] + x
                ab[i, :] = x * d
            pltpu.async_copy(sb, sum_f.at[pl.ds(hoff + g0, EW), :], semw)
            pltpu.async_copy(ab, y_f.at[pl.ds(hoff + g0, EW), :], semw)
        else:
            @pl.loop(0, EW)
            def _(i):
                x = ab[i, :] * _bcast16(dw, i)
                sb[i, :] = (sb[i, :] + x) * 0.25
            pltpu.async_copy(sb, out.at[pl.ds(hoff + g0, EW), :], semw)

    def _epi_wait_w(l, ab, sb, semw):
        if l < NLAYERS - 1:
            pltpu.make_async_copy(sum_f.at[pl.ds(0, EW), :], sb,
                                  semw).wait()
            pltpu.make_async_copy(sum_f.at[pl.ds(0, EW), :], ab,
                                  semw).wait()
        else:
            pltpu.make_async_copy(sum_f.at[pl.ds(0, EW), :], sb,
                                  semw).wait()

    for l in range(NLAYERS):
        _idx_fire(0, 0)
        _idx_fire(1, 1)
        _idx_fire(2, 2)
        _idx_wait(0)
        _gather_fire(0, 0)
        _idx_wait(1)
        _gather_fire(1, 1)

        @pl.loop(0, NWIN, step=6)
        def _(w):
            for u in range(6):
                b, p = u % 3, u % 2
                wu = w + u
                _drain(b, p)

                @pl.when(wu + 3 < NWIN)
                def _():
                    _idx_fire(wu + 3, b)

                @pl.when(wu + 2 < NWIN)
                def _():
                    _idx_wait((u + 2) % 3)
                    _gather_fire((u + 2) % 3, p)

        plsc.subcore_barrier()

        @pl.loop(0, CHUNK, step=EW)
        def _(w):
            g0 = r0 + w
            pltpu.sync_copy(accum.at[pl.ds(g0, EW), :], ab0)
            pltpu.sync_copy(zb, accum.at[pl.ds(g0, EW), :])
            pltpu.sync_copy(dd.at[pl.ds(g0, EW)], dw0)
            if l == 0:
                pltpu.sync_copy(emb_f.at[pl.ds(hoff + g0, EW), :], sb0)
            else:
                pltpu.sync_copy(sum_f.at[pl.ds(hoff + g0, EW), :], sb0)

            if l < NLAYERS - 1:
                @pl.loop(0, EW)
                def _(i):
                    d = _bcast16(dw0, i)
                    x = ab0[i, :] * d
                    sb0[i, :] = sb0[i, :] + x
                    ab0[i, :] = x * d
                pltpu.sync_copy(sb0, sum_f.at[pl.ds(hoff + g0, EW), :])
                pltpu.sync_copy(ab0, y_f.at[pl.ds(hoff + g0, EW), :])
            else:
                @pl.loop(0, EW)
                def _(i):
                    x = ab0[i, :] * _bcast16(dw0, i)
                    sb0[i, :] = (sb0[i, :] + x) * 0.25
                pltpu.sync_copy(sb0, out.at[pl.ds(hoff + g0, EW), :])

        plsc.subcore_barrier()


@jax.jit
def _lightgcn_sc(emb_f, rc3):
    cp = pltpu.CompilerParams(use_tc_tiling_on_sc=False)
    if "needs_layout_passes" in pltpu.CompilerParams.__dataclass_fields__:
        cp = dataclasses.replace(cp, needs_layout_passes=False)
    mesh = plsc.VectorSubcoreMesh(core_axis_name="c", subcore_axis_name="s")
    k = pl.kernel(
        _sc_body,
        out_type=jax.ShapeDtypeStruct((2 * NP, H), jnp.float32),
        mesh=mesh,
        scratch_types=[
            pltpu.HBM((2 * NP, H), jnp.float32),
            pltpu.HBM((2 * NP, H), jnp.float32),
            pltpu.HBM((EW,), jnp.float32),
            pltpu.VMEM_SHARED((NP, H), jnp.float32),
            pltpu.VMEM_SHARED((NP,), jnp.float32),
            pltpu.VMEM((EW, H), jnp.float32),
            pltpu.VMEM((128,), jnp.float32),
            pltpu.VMEM((EW, H), jnp.float32),
            pltpu.VMEM((EW, H), jnp.float32),
            pltpu.VMEM((EW,), jnp.float32),
            pltpu.VMEM((EW,), jnp.float32),
            pltpu.VMEM((EW, H), jnp.float32),
            pltpu.VMEM((EW, H), jnp.float32),
            pltpu.VMEM((6, 128), jnp.int32),
            pltpu.VMEM((6, 128), jnp.int32),
            pltpu.VMEM((6, 128), jnp.int32),
            pltpu.VMEM((WSZ, H), jnp.float32),
            pltpu.VMEM((WSZ, H), jnp.float32),
            pltpu.SemaphoreType.DMA,
            pltpu.SemaphoreType.DMA,
            pltpu.SemaphoreType.DMA,
            pltpu.SemaphoreType.DMA,
            pltpu.SemaphoreType.DMA,
            pltpu.SemaphoreType.DMA,
            pltpu.SemaphoreType.DMA,
            pltpu.SemaphoreType.DMA,
            pltpu.SemaphoreType.DMA,
        ],
        compiler_params=cp,
    )
    return k(emb_f, rc3)


def kernel(emb, edge_index):
    emb = emb.astype(jnp.float32)
    row = edge_index[0].astype(jnp.int32)
    col = edge_index[1].astype(jnp.int32)
    npad = EPAD - E
    ar = jnp.arange(npad, dtype=jnp.int32)
    pad_idx = N + ar % (NP - N)
    row_w = jnp.concatenate([row, pad_idx]).reshape(16, NWIN, 3, 128)
    col_w = jnp.concatenate([col, pad_idx]).reshape(16, NWIN, 3, 128)
    rc3 = jnp.stack([
        jnp.concatenate([col_w + off, row_w], axis=2).reshape(-1, 128)
        for off in (0, NP)
    ])
    emb_f = jnp.zeros((2 * NP, H), jnp.float32)
    emb_f = emb_f.at[:N].set(emb[:, :H]).at[NP:NP + N].set(emb[:, H:])
    outp = _lightgcn_sc(emb_f, rc3)
    final = jnp.concatenate([outp[:N], outp[NP:NP + N]], axis=1)
    return final[:40000], final[40000:90000], final[90000:]


# speedup vs baseline: 28.8749x; 1.0118x over previous
import dataclasses

import jax
import jax.numpy as jnp
from jax import lax
from jax.experimental import pallas as pl
from jax.experimental.pallas import tpu as pltpu
from jax.experimental.pallas import tpu_sc as plsc

N = 100000
H = 16
NP = 100352
E = 1600000
WSZ = 384
NWIN = 264
EPT = WSZ * NWIN
EPAD = EPT * 16
CHUNK = NP // 16
EW = 112
NLAYERS = 3


def _rsqrt16(d):
    i = plsc.bitcast(d, jnp.int32)
    i = jnp.int32(0x5F3759DF) - lax.shift_right_logical(i, 1)
    y = plsc.bitcast(i, jnp.float32)
    for _ in range(3):
        y = y * (1.5 - 0.5 * d * y * y)
    return jnp.where(d > 0.0, y, 0.0)


def _bcast16(ref, i):
    return plsc.load_gather(ref, [jnp.full((16,), i, jnp.int32)])


def _sc_body(emb_f, rc3, out,
             y_f, sum_f, hdum,
             accum, dd,
             zb, ones,
             ab0, ab1, dw0, dw1, sb0, sb1,
             ic0, ic1, ic2, v0, v1,
             semA, semB, semI0, semI1, semI2,
             semEA, semEB, semWA, semWB):
    c = lax.axis_index("c")
    t = lax.axis_index("s")
    r0 = t * CHUNK
    i0 = t * (NWIN * 2)
    hoff = c * NP

    ics = (ic0, ic1, ic2)
    semIs = (semI0, semI1, semI2)
    vs = (v0, v1)
    semVs = (semA, semB)

    z16 = jnp.zeros((16,), jnp.float32)

    @pl.loop(0, EW)
    def _(i):
        zb[i, :] = z16

    @pl.loop(0, EW, step=16)
    def _(i):
        dw0[pl.ds(i, 16)] = z16

    @pl.loop(0, WSZ, step=16)
    def _(i):
        ones[pl.ds(i, 16)] = jnp.ones((16,), jnp.float32)

    @pl.loop(0, CHUNK, step=EW)
    def _(w):
        pltpu.sync_copy(zb, accum.at[pl.ds(r0 + w, EW), :])
        pltpu.sync_copy(dw0, dd.at[pl.ds(r0 + w, EW)])

    plsc.subcore_barrier()

    def _idx_fire(w, b):
        pltpu.async_copy(rc3.at[c, pl.ds(i0 + w * 2, 2), :], ics[b],
                         semIs[b])

    def _idx_wait(b):
        pltpu.make_async_copy(rc3.at[c, pl.ds(0, 2), :], ics[b],
                              semIs[b]).wait()

    _idx_fire(0, 0)

    @pl.loop(0, NWIN, step=2)
    def _(w):
        _idx_fire(w + 1, 1)
        _idx_wait(0)
        pltpu.sync_copy(ones, dd.at[ic0.at[1]], add=True)

        @pl.when(w + 2 < NWIN)
        def _():
            _idx_fire(w + 2, 0)

        _idx_wait(1)
        pltpu.sync_copy(ones, dd.at[ic1.at[1]], add=True)

    plsc.subcore_barrier()

    @pl.loop(0, CHUNK, step=EW)
    def _(w):
        g0 = r0 + w
        pltpu.sync_copy(dd.at[pl.ds(g0, EW)], dw0)

        @pl.loop(0, EW, step=16)
        def _(i):
            dw0[pl.ds(i, 16)] = _rsqrt16(dw0[pl.ds(i, 16)])

        pltpu.sync_copy(dw0, dd.at[pl.ds(g0, EW)])
        pltpu.sync_copy(emb_f.at[pl.ds(hoff + g0, EW), :], ab0)

        @pl.loop(0, EW)
        def _(i):
            ab0[i, :] = ab0[i, :] * _bcast16(dw0, i)

        pltpu.sync_copy(ab0, y_f.at[pl.ds(hoff + g0, EW), :])

    plsc.subcore_barrier()

    def _gather_fire(b, p):
        pltpu.async_copy(y_f.at[ics[b].at[0]], vs[p], semVs[p])

    def _drain(b, p):
        pltpu.make_async_copy(y_f.at[ics[b].at[0]], vs[p],
                              semVs[p]).wait()
        pltpu.sync_copy(vs[p], accum.at[ics[b].at[1]], add=True)

    def _epi_fire_in(l, w, ab, dw, sb, sem):
        g0 = r0 + w
        pltpu.async_copy(accum.at[pl.ds(g0, EW), :], ab, sem)
        pltpu.async_copy(dd.at[pl.ds(g0, EW)], dw, sem)
        src = emb_f if l == 0 else sum_f
        pltpu.async_copy(src.at[pl.ds(hoff + g0, EW), :], sb, sem)

    def _epi_wait_in(l, ab, dw, sb, sem):
        pltpu.make_async_copy(sum_f.at[pl.ds(0, EW), :], ab, sem).wait()
        pltpu.make_async_copy(hdum, dw, sem).wait()
        pltpu.make_async_copy(sum_f.at[pl.ds(0, EW), :], sb, sem).wait()

    def _epi_compute_write(l, w, ab, dw, sb, semw):
        g0 = r0 + w
        pltpu.sync_copy(zb, accum.at[pl.ds(g0, EW), :])
        if l < NLAYERS - 1:
            @pl.loop(0, EW)
            def _(i):
                d = _bcast16(dw, i)
                x = ab[i, :] * d
                sb[i, :] = sb[i, :] + x
                ab[i, :] = x * d
            pltpu.async_copy(sb, sum_f.at[pl.ds(hoff + g0, EW), :], semw)
            pltpu.async_copy(ab, y_f.at[pl.ds(hoff + g0, EW), :], semw)
        else:
            @pl.loop(0, EW)
            def _(i):
                x = ab[i, :] * _bcast16(dw, i)
                sb[i, :] = (sb[i, :] + x) * 0.25
            pltpu.async_copy(sb, out.at[pl.ds(hoff + g0, EW), :], semw)

    def _epi_wait_w(l, ab, sb, semw):
        if l < NLAYERS - 1:
            pltpu.make_async_copy(sum_f.at[pl.ds(0, EW), :], sb,
                                  semw).wait()
            pltpu.make_async_copy(sum_f.at[pl.ds(0, EW), :], ab,
                                  semw).wait()
        else:
            pltpu.make_async_copy(sum_f.at[pl.ds(0, EW), :], sb,
                                  semw).wait()

    for l in range(NLAYERS):
        _idx_fire(0, 0)
        _idx_fire(1, 1)
        _idx_fire(2, 2)
        _idx_wait(0)
        _gather_fire(0, 0)
        _idx_wait(1)
        _gather_fire(1, 1)

        @pl.loop(0, NWIN, step=6)
        def _(w):
            for u in range(6):
                b, p = u % 3, u % 2
                wu = w + u
                _drain(b, p)

                @pl.when(wu + 3 < NWIN)
                def _():
                    _idx_fire(wu + 3, b)

                @pl.when(wu + 2 < NWIN)
                def _():
                    _idx_wait((u + 2) % 3)
                    _gather_fire((u + 2) % 3, p)

        plsc.subcore_barrier()

        @pl.loop(0, CHUNK, step=EW)
        def _(w):
            g0 = r0 + w
            pltpu.sync_copy(accum.at[pl.ds(g0, EW), :], ab0)
            pltpu.sync_copy(zb, accum.at[pl.ds(g0, EW), :])
            pltpu.sync_copy(dd.at[pl.ds(g0, EW)], dw0)
            if l == 0:
                pltpu.sync_copy(emb_f.at[pl.ds(hoff + g0, EW), :], sb0)
            else:
                pltpu.sync_copy(sum_f.at[pl.ds(hoff + g0, EW), :], sb0)

            if l < NLAYERS - 1:
                @pl.loop(0, EW)
                def _(i):
                    d = _bcast16(dw0, i)
                    x = ab0[i, :] * d
                    sb0[i, :] = sb0[i, :] + x
                    ab0[i, :] = x * d
                pltpu.sync_copy(sb0, sum_f.at[pl.ds(hoff + g0, EW), :])
                pltpu.sync_copy(ab0, y_f.at[pl.ds(hoff + g0, EW), :])
            else:
                @pl.loop(0, EW)
                def _(i):
                    x = ab0[i, :] * _bcast16(dw0, i)
                    sb0[i, :] = (sb0[i, :] + x) * 0.25
                pltpu.sync_copy(sb0, out.at[pl.ds(hoff + g0, EW), :])

        plsc.subcore_barrier()


@jax.jit
def _lightgcn_sc(emb_f, rc3):
    cp = pltpu.CompilerParams(use_tc_tiling_on_sc=False)
    if "needs_layout_passes" in pltpu.CompilerParams.__dataclass_fields__:
        cp = dataclasses.replace(cp, needs_layout_passes=False)
    mesh = plsc.VectorSubcoreMesh(core_axis_name="c", subcore_axis_name="s")
    k = pl.kernel(
        _sc_body,
        out_type=jax.ShapeDtypeStruct((2 * NP, H), jnp.float32),
        mesh=mesh,
        scratch_types=[
            pltpu.HBM((2 * NP, H), jnp.float32),
            pltpu.HBM((2 * NP, H), jnp.float32),
            pltpu.HBM((EW,), jnp.float32),
            pltpu.VMEM_SHARED((NP, H), jnp.float32),
            pltpu.VMEM_SHARED((NP,), jnp.float32),
            pltpu.VMEM((EW, H), jnp.float32),
            pltpu.VMEM((WSZ,), jnp.float32),
            pltpu.VMEM((EW, H), jnp.float32),
            pltpu.VMEM((EW, H), jnp.float32),
            pltpu.VMEM((EW,), jnp.float32),
            pltpu.VMEM((EW,), jnp.float32),
            pltpu.VMEM((EW, H), jnp.float32),
            pltpu.VMEM((EW, H), jnp.float32),
            pltpu.VMEM((2, WSZ), jnp.int32),
            pltpu.VMEM((2, WSZ), jnp.int32),
            pltpu.VMEM((2, WSZ), jnp.int32),
            pltpu.VMEM((WSZ, H), jnp.float32),
            pltpu.VMEM((WSZ, H), jnp.float32),
            pltpu.SemaphoreType.DMA,
            pltpu.SemaphoreType.DMA,
            pltpu.SemaphoreType.DMA,
            pltpu.SemaphoreType.DMA,
            pltpu.SemaphoreType.DMA,
            pltpu.SemaphoreType.DMA,
            pltpu.SemaphoreType.DMA,
            pltpu.SemaphoreType.DMA,
            pltpu.SemaphoreType.DMA,
        ],
        compiler_params=cp,
    )
    return k(emb_f, rc3)


def kernel(emb, edge_index):
    emb = emb.astype(jnp.float32)
    row = edge_index[0].astype(jnp.int32)
    col = edge_index[1].astype(jnp.int32)
    npad = EPAD - E
    ar = jnp.arange(npad, dtype=jnp.int32)
    pad_idx = N + ar % (NP - N)
    row_w = jnp.concatenate([row, pad_idx]).reshape(16, NWIN, 1, WSZ)
    col_w = jnp.concatenate([col, pad_idx]).reshape(16, NWIN, 1, WSZ)
    rc3 = jnp.stack([
        jnp.concatenate([col_w + off, row_w], axis=2).reshape(-1, WSZ)
        for off in (0, NP)
    ])
    emb_f = jnp.zeros((2 * NP, H), jnp.float32)
    emb_f = emb_f.at[:N].set(emb[:, :H]).at[NP:NP + N].set(emb[:, H:])
    outp = _lightgcn_sc(emb_f, rc3)
    final = jnp.concatenate([outp[:N], outp[NP:NP + N]], axis=1)
    return final[:40000], final[40000:90000], final[90000:]


# speedup vs baseline: 32.4076x; 1.1223x over previous
import dataclasses

import jax
import jax.numpy as jnp
from jax import lax
from jax.experimental import pallas as pl
from jax.experimental.pallas import tpu as pltpu
from jax.experimental.pallas import tpu_sc as plsc

N = 100000
H = 16
NP = 100352
E = 1600000
WSZ = 384
NWIN = 264
EPT = WSZ * NWIN
EPAD = EPT * 16
CHUNK = NP // 16
EW = 112
NLAYERS = 3


def _rsqrt16(d):
    i = plsc.bitcast(d, jnp.int32)
    i = jnp.int32(0x5F3759DF) - lax.shift_right_logical(i, 1)
    y = plsc.bitcast(i, jnp.float32)
    for _ in range(3):
        y = y * (1.5 - 0.5 * d * y * y)
    return jnp.where(d > 0.0, y, 0.0)


def _bcast16(ref, i):
    return plsc.load_gather(ref, [jnp.full((16,), i, jnp.int32)])


def _sc_body(emb_f, rc3, out,
             y_f, sum_f, hdum,
             accum, dd,
             zb, ones,
             ab0, ab1, dw0, dw1, sb0, sb1,
             ic0, ic1, ic2, v0, v1,
             semA, semB, semI0, semI1, semI2,
             semEA, semEB, semWA, semWB):
    c = lax.axis_index("c")
    t = lax.axis_index("s")
    r0 = t * CHUNK
    i0 = t * (NWIN * 2)
    hoff = c * NP

    ics = (ic0, ic1, ic2)
    semIs = (semI0, semI1, semI2)
    vs = (v0, v1)
    semVs = (semA, semB)

    z16 = jnp.zeros((16,), jnp.float32)

    @pl.loop(0, EW)
    def _(i):
        zb[i, :] = z16

    @pl.loop(0, EW, step=16)
    def _(i):
        dw0[pl.ds(i, 16)] = z16

    @pl.loop(0, WSZ, step=16)
    def _(i):
        ones[pl.ds(i, 16)] = jnp.ones((16,), jnp.float32)

    @pl.loop(0, CHUNK, step=EW)
    def _(w):
        pltpu.sync_copy(zb, accum.at[pl.ds(r0 + w, EW), :])
        pltpu.sync_copy(dw0, dd.at[pl.ds(r0 + w, EW)])

    plsc.subcore_barrier()

    def _idx_fire(w, b):
        pltpu.async_copy(rc3.at[pl.ds(i0 + w * 2, 2), :], ics[b],
                         semIs[b])

    def _idx_wait(b):
        pltpu.make_async_copy(rc3.at[pl.ds(0, 2), :], ics[b],
                              semIs[b]).wait()

    _idx_fire(0, 0)

    @pl.loop(0, NWIN, step=2)
    def _(w):
        _idx_fire(w + 1, 1)
        _idx_wait(0)
        pltpu.sync_copy(ones, dd.at[ic0.at[1]], add=True)

        @pl.when(w + 2 < NWIN)
        def _():
            _idx_fire(w + 2, 0)

        _idx_wait(1)
        pltpu.sync_copy(ones, dd.at[ic1.at[1]], add=True)

    plsc.subcore_barrier()

    @pl.loop(0, CHUNK, step=EW)
    def _(w):
        g0 = r0 + w
        pltpu.sync_copy(dd.at[pl.ds(g0, EW)], dw0)

        @pl.loop(0, EW, step=16)
        def _(i):
            dw0[pl.ds(i, 16)] = _rsqrt16(dw0[pl.ds(i, 16)])

        pltpu.sync_copy(dw0, dd.at[pl.ds(g0, EW)])
        pltpu.sync_copy(emb_f.at[pl.ds(hoff + g0, EW), :], ab0)

        @pl.loop(0, EW)
        def _(i):
            ab0[i, :] = ab0[i, :] * _bcast16(dw0, i)

        pltpu.sync_copy(ab0, y_f.at[pl.ds(hoff + g0, EW), :])

    plsc.subcore_barrier()

    def _gather_fire(b, p):
        @pl.loop(0, WSZ, step=16)
        def _(i):
            ics[b][0, pl.ds(i, 16)] = ics[b][0, pl.ds(i, 16)] + hoff
        pltpu.async_copy(y_f.at[ics[b].at[0]], vs[p], semVs[p])

    def _drain(b, p):
        pltpu.make_async_copy(y_f.at[ics[b].at[0]], vs[p],
                              semVs[p]).wait()
        pltpu.sync_copy(vs[p], accum.at[ics[b].at[1]], add=True)

    def _epi_fire_in(l, w, ab, dw, sb, sem):
        g0 = r0 + w
        pltpu.async_copy(accum.at[pl.ds(g0, EW), :], ab, sem)
        pltpu.async_copy(dd.at[pl.ds(g0, EW)], dw, sem)
        src = emb_f if l == 0 else sum_f
        pltpu.async_copy(src.at[pl.ds(hoff + g0, EW), :], sb, sem)

    def _epi_wait_in(l, ab, dw, sb, sem):
        pltpu.make_async_copy(sum_f.at[pl.ds(0, EW), :], ab, sem).wait()
        pltpu.make_async_copy(hdum, dw, sem).wait()
        pltpu.make_async_copy(sum_f.at[pl.ds(0, EW), :], sb, sem).wait()

    def _epi_compute_write(l, w, ab, dw, sb, semw):
        g0 = r0 + w
        pltpu.sync_copy(zb, accum.at[pl.ds(g0, EW), :])
        if l < NLAYERS - 1:
            @pl.loop(0, EW)
            def _(i):
                d = _bcast16(dw, i)
                x = ab[i, :] * d
                sb[i, :] = sb[i, :] + x
                ab[i, :] = x * d
            pltpu.async_copy(sb, sum_f.at[pl.ds(hoff + g0, EW), :], semw)
            pltpu.async_copy(ab, y_f.at[pl.ds(hoff + g0, EW), :], semw)
        else:
            @pl.loop(0, EW)
            def _(i):
                x = ab[i, :] * _bcast16(dw, i)
                sb[i, :] = (sb[i, :] + x) * 0.25
            pltpu.async_copy(sb, out.at[pl.ds(hoff + g0, EW), :], semw)

    def _epi_wait_w(l, ab, sb, semw):
        if l < NLAYERS - 1:
            pltpu.make_async_copy(sum_f.at[pl.ds(0, EW), :], sb,
                                  semw).wait()
            pltpu.make_async_copy(sum_f.at[pl.ds(0, EW), :], ab,
                                  semw).wait()
        else:
            pltpu.make_async_copy(sum_f.at[pl.ds(0, EW), :], sb,
                                  semw).wait()

    for l in range(NLAYERS):
        _idx_fire(0, 0)
        _idx_fire(1, 1)
        _idx_fire(2, 2)
        _idx_wait(0)
        _gather_fire(0, 0)
        _idx_wait(1)
        _gather_fire(1, 1)

        @pl.loop(0, NWIN, step=6)
        def _(w):
            for u in range(6):
                b, p = u % 3, u % 2
                wu = w + u
                _drain(b, p)

                @pl.when(wu + 3 < NWIN)
                def _():
                    _idx_fire(wu + 3, b)

                @pl.when(wu + 2 < NWIN)
                def _():
                    _idx_wait((u + 2) % 3)
                    _gather_fire((u + 2) % 3, p)

        plsc.subcore_barrier()

        @pl.loop(0, CHUNK, step=EW)
        def _(w):
            g0 = r0 + w
            pltpu.sync_copy(accum.at[pl.ds(g0, EW), :], ab0)
            pltpu.sync_copy(zb, accum.at[pl.ds(g0, EW), :])
            pltpu.sync_copy(dd.at[pl.ds(g0, EW)], dw0)
            if l == 0:
                pltpu.sync_copy(emb_f.at[pl.ds(hoff + g0, EW), :], sb0)
            else:
                pltpu.sync_copy(sum_f.at[pl.ds(hoff + g0, EW), :], sb0)

            if l < NLAYERS - 1:
                @pl.loop(0, EW)
                def _(i):
                    d = _bcast16(dw0, i)
                    x = ab0[i, :] * d
                    sb0[i, :] = sb0[i, :] + x
                    ab0[i, :] = x * d
                pltpu.sync_copy(sb0, sum_f.at[pl.ds(hoff + g0, EW), :])
                pltpu.sync_copy(ab0, y_f.at[pl.ds(hoff + g0, EW), :])
            else:
                @pl.loop(0, EW)
                def _(i):
                    x = ab0[i, :] * _bcast16(dw0, i)
                    sb0[i, :] = (sb0[i, :] + x) * 0.25
                pltpu.sync_copy(
                    sb0, out.at[pl.ds(g0, EW), pl.ds(c * H, H)])

        plsc.subcore_barrier()


@jax.jit
def _lightgcn_sc(emb_f, rc3):
    cp = pltpu.CompilerParams(use_tc_tiling_on_sc=False)
    if "needs_layout_passes" in pltpu.CompilerParams.__dataclass_fields__:
        cp = dataclasses.replace(cp, needs_layout_passes=False)
    mesh = plsc.VectorSubcoreMesh(core_axis_name="c", subcore_axis_name="s")
    k = pl.kernel(
        _sc_body,
        out_type=jax.ShapeDtypeStruct((NP, 2 * H), jnp.float32),
        mesh=mesh,
        scratch_types=[
            pltpu.HBM((2 * NP, H), jnp.float32),
            pltpu.HBM((2 * NP, H), jnp.float32),
            pltpu.HBM((EW,), jnp.float32),
            pltpu.VMEM_SHARED((NP, H), jnp.float32),
            pltpu.VMEM_SHARED((NP,), jnp.float32),
            pltpu.VMEM((EW, H), jnp.float32),
            pltpu.VMEM((WSZ,), jnp.float32),
            pltpu.VMEM((EW, H), jnp.float32),
            pltpu.VMEM((EW, H), jnp.float32),
            pltpu.VMEM((EW,), jnp.float32),
            pltpu.VMEM((EW,), jnp.float32),
            pltpu.VMEM((EW, H), jnp.float32),
            pltpu.VMEM((EW, H), jnp.float32),
            pltpu.VMEM((2, WSZ), jnp.int32),
            pltpu.VMEM((2, WSZ), jnp.int32),
            pltpu.VMEM((2, WSZ), jnp.int32),
            pltpu.VMEM((WSZ, H), jnp.float32),
            pltpu.VMEM((WSZ, H), jnp.float32),
            pltpu.SemaphoreType.DMA,
            pltpu.SemaphoreType.DMA,
            pltpu.SemaphoreType.DMA,
            pltpu.SemaphoreType.DMA,
            pltpu.SemaphoreType.DMA,
            pltpu.SemaphoreType.DMA,
            pltpu.SemaphoreType.DMA,
            pltpu.SemaphoreType.DMA,
            pltpu.SemaphoreType.DMA,
        ],
        compiler_params=cp,
    )
    return k(emb_f, rc3)


def kernel(emb, edge_index):
    emb = emb.astype(jnp.float32)
    row = edge_index[0].astype(jnp.int32)
    col = edge_index[1].astype(jnp.int32)
    npad = EPAD - E
    ar = jnp.arange(npad, dtype=jnp.int32)
    pad_idx = N + ar % (NP - N)
    row_w = jnp.concatenate([row, pad_idx]).reshape(16, NWIN, 1, WSZ)
    col_w = jnp.concatenate([col, pad_idx]).reshape(16, NWIN, 1, WSZ)
    rc3 = jnp.concatenate([col_w, row_w], axis=2).reshape(-1, WSZ)
    emb_f = jnp.zeros((2 * NP, H), jnp.float32)
    emb_f = emb_f.at[:N].set(emb[:, :H]).at[NP:NP + N].set(emb[:, H:])
    final = _lightgcn_sc(emb_f, rc3)
    return final[:40000], final[40000:90000], final[90000:N]
